# baseline (speedup 1.0000x reference)
import jax
import jax.numpy as jnp
from jax.experimental import pallas as pl
from jax.experimental.pallas import tpu as pltpu
from jax.experimental.pallas import tpu_sc as plsc

L = 200
L_PAD = 256
D = 64
DP = 128
WINDOW = 128
V2 = 1000


def _sc_gather(e0, e1, e2, j0, j1, j2):
    mesh = plsc.VectorSubcoreMesh(core_axis_name="core",
                                  subcore_axis_name="subcore")
    out_t = tuple(jax.ShapeDtypeStruct((L_PAD, DP), jnp.float32)
                  for _ in range(3))

    @pl.kernel(out_type=out_t, mesh=mesh)
    def k(e0_hbm, e1_hbm, e2_hbm, j0_hbm, j1_hbm, j2_hbm,
          o0_hbm, o1_hbm, o2_hbm):
        for emb_hbm, idx_hbm, out_hbm in ((e0_hbm, j0_hbm, o0_hbm),
                                          (e1_hbm, j1_hbm, o1_hbm),
                                          (e2_hbm, j2_hbm, o2_hbm)):
            def body(i_vmem, o_vmem, emb=emb_hbm):
                pltpu.sync_copy(emb.at[i_vmem.at[0]], o_vmem)

            pltpu.emit_pipeline(
                body,
                grid=(L_PAD // WINDOW,),
                in_specs=[pl.BlockSpec((1, WINDOW), index_map=lambda i: (i, 0))],
                out_specs=[pl.BlockSpec((WINDOW, DP), index_map=lambda i: (i, 0))],
                core_axis_name="subcore",
                dimension_semantics=(pltpu.PARALLEL,),
            )(idx_hbm, out_hbm)

    return k(e0, e1, e2, j0, j1, j2)


def _pick(g_ref, par_ref):
    par = par_ref[:L, :]
    lo = g_ref[:L, :D] * (1.0 - par)
    hi = g_ref[:L, D:] * par
    return jnp.sum(lo + hi, axis=0, keepdims=True)


def _dense_body(g0_ref, g1_ref, g2_ref, p0_ref, p1_ref, p2_ref,
                w_ref, b_ref, ddi_ref, res_ref, bn_ref):
    i1 = _pick(g0_ref, p0_ref)
    i2 = _pick(g1_ref, p1_ref)
    i3 = _pick(g2_ref, p2_ref)
    x = jnp.concatenate([i1, i2, i3], axis=1)
    x = jnp.maximum(x, 0.0)
    r = jnp.dot(x, w_ref[...],
                preferred_element_type=jnp.float32,
                precision=jax.lax.Precision.HIGHEST) + b_ref[...]
    res_ref[...] = r
    p = jax.nn.sigmoid(r)
    v = jnp.dot(p, ddi_ref[...],
                preferred_element_type=jnp.float32,
                precision=jax.lax.Precision.HIGHEST)
    bn_ref[...] = (0.0005 * jnp.sum(v * p))[None, None]


def _dense(g0, g1, g2, p0, p1, p2, w_t, b2, ddi):
    return pl.pallas_call(
        _dense_body,
        out_shape=(jax.ShapeDtypeStruct((1, V2), jnp.float32),
                   jax.ShapeDtypeStruct((1, 1), jnp.float32)),
    )(g0, g1, g2, p0, p1, p2, w_t, b2, ddi)


def _prep_idx(idx):
    idx = idx.astype(jnp.int32).reshape(-1)
    idx = jnp.pad(idx, (0, L_PAD - L))
    pair = (idx // 2).reshape(2, WINDOW)
    par = (idx % 2).astype(jnp.float32).reshape(L_PAD, 1)
    return pair, par


def kernel(diag_idx, proc_idx, med_idx, emb0, emb1, emb2, W, b, ddi_adj):
    j0, p0 = _prep_idx(diag_idx)
    j1, p1 = _prep_idx(proc_idx)
    j2, p2 = _prep_idx(med_idx)
    e0 = emb0.reshape(-1, DP)
    e1 = emb1.reshape(-1, DP)
    e2 = emb2.reshape(-1, DP)
    g0, g1, g2 = _sc_gather(e0, e1, e2, j0, j1, j2)
    res, bn = _dense(g0, g1, g2, p0, p1, p2,
                     W.T, b.reshape(1, V2), ddi_adj)
    return res, bn[0, 0]


# speedup vs baseline: 1.5167x; 1.5167x over previous
import jax
import jax.numpy as jnp
from jax import lax
from jax.experimental import pallas as pl
from jax.experimental.pallas import tpu as pltpu
from jax.experimental.pallas import tpu_sc as plsc

L = 200
L_PAD = 256
D = 64
V2 = 1000


def _sc_gather(e0, e1, j0, j1):
    mesh = plsc.ScalarSubcoreMesh(axis_name="core", num_cores=2)
    out_t = tuple(jax.ShapeDtypeStruct((L_PAD, D), jnp.float32)
                  for _ in range(2))

    @pl.kernel(out_type=out_t, mesh=mesh,
               scratch_types=[pltpu.SMEM((L_PAD,), jnp.int32),
                              pltpu.SemaphoreType.DMA,
                              pltpu.SemaphoreType.DMA])
    def k(e0_hbm, e1_hbm, j0_hbm, j1_hbm, o0_hbm, o1_hbm,
          idx_s, sem_i, sem_g):
        core = lax.axis_index("core")

        def gather(emb_hbm, idx_hbm, out_hbm):
            pltpu.async_copy(idx_hbm, idx_s, sem_i).wait()

            @pl.loop(0, L)
            def _(i):
                pltpu.make_async_copy(emb_hbm.at[idx_s[i]],
                                      out_hbm.at[i], sem_g).start()

            @pl.loop(0, L)
            def _(i):
                pltpu.make_async_copy(emb_hbm.at[0],
                                      out_hbm.at[i], sem_g).wait()

        @pl.when(core == 0)
        def _():
            gather(e0_hbm, j0_hbm, o0_hbm)

        @pl.when(core == 1)
        def _():
            gather(e1_hbm, j1_hbm, o1_hbm)

    return k(e0, e1, j0, j1)


def _dense_body(g0_ref, g1_ref, med_ref, e2_ref, w_ref, b_ref, ddi_ref,
                res_ref, bn_ref):
    i1 = jnp.sum(g0_ref[:L, :], axis=0, keepdims=True)
    i2 = jnp.sum(g1_ref[:L, :], axis=0, keepdims=True)
    iota = lax.broadcasted_iota(jnp.int32, (1, V2), 1)
    eq = (med_ref[...] == iota).astype(jnp.float32)
    counts = jnp.sum(eq, axis=0, keepdims=True)
    i3 = jnp.dot(counts, e2_ref[...],
                 preferred_element_type=jnp.float32,
                 precision=lax.Precision.HIGHEST)
    x = jnp.concatenate([i1, i2, i3], axis=1)
    x = jnp.maximum(x, 0.0)
    r = lax.dot_general(
        x, w_ref[...], (((1,), (1,)), ((), ())),
        preferred_element_type=jnp.float32,
        precision=lax.Precision.HIGHEST) + b_ref[...]
    res_ref[...] = r
    p = jax.nn.sigmoid(r)
    v = jnp.dot(p, ddi_ref[...],
                preferred_element_type=jnp.float32,
                precision=lax.Precision.HIGHEST)
    bn_ref[...] = (0.0005 * jnp.sum(v * p))[None, None]


def _dense(g0, g1, med_col, e2, w, b2, ddi):
    return pl.pallas_call(
        _dense_body,
        out_shape=(jax.ShapeDtypeStruct((1, V2), jnp.float32),
                   jax.ShapeDtypeStruct((1, 1), jnp.float32)),
    )(g0, g1, med_col, e2, w, b2, ddi)


def _prep_idx(idx, fill=0):
    idx = idx.astype(jnp.int32).reshape(-1)
    return jnp.pad(idx, (0, L_PAD - L), constant_values=fill)


def kernel(diag_idx, proc_idx, med_idx, emb0, emb1, emb2, W, b, ddi_adj):
    j0 = _prep_idx(diag_idx)
    j1 = _prep_idx(proc_idx)
    med_col = _prep_idx(med_idx, fill=-1).reshape(L_PAD, 1)
    g0, g1 = _sc_gather(emb0, emb1, j0, j1)
    res, bn = _dense(g0, g1, med_col, emb2, W, b.reshape(1, V2), ddi_adj)
    return res, bn[0, 0]


# speedup vs baseline: 1.7483x; 1.1528x over previous
import jax
import jax.numpy as jnp
from jax import lax
from jax.experimental import pallas as pl
from jax.experimental.pallas import tpu as pltpu

L = 200
L_PAD = 256
D = 64
V2 = 1000


def _body(i0_ref, i1_ref, med_ref, e0_hbm, e1_hbm, e2_ref, w_ref, b_ref,
          ddi_ref, res_ref, bn_ref, rows0, rows1, sem0, sem1):
    def fire(i, _):
        pltpu.make_async_copy(e0_hbm.at[i0_ref[0, i]], rows0.at[i],
                              sem0).start()
        pltpu.make_async_copy(e1_hbm.at[i1_ref[0, i]], rows1.at[i],
                              sem1).start()
        return _
    lax.fori_loop(0, L, fire, None)

    iota = lax.broadcasted_iota(jnp.int32, (1, V2), 1)
    eq = (med_ref[...] == iota).astype(jnp.float32)
    counts = jnp.sum(eq, axis=0, keepdims=True)
    i3 = jnp.dot(counts, e2_ref[...],
                 preferred_element_type=jnp.float32,
                 precision=lax.Precision.HIGHEST)

    def drain(i, _):
        pltpu.make_async_copy(e0_hbm.at[0], rows0.at[i], sem0).wait()
        pltpu.make_async_copy(e1_hbm.at[0], rows1.at[i], sem1).wait()
        return _
    lax.fori_loop(0, L, drain, None)

    i1 = jnp.sum(rows0[:L, :], axis=0, keepdims=True)
    i2 = jnp.sum(rows1[:L, :], axis=0, keepdims=True)
    x = jnp.concatenate([i1, i2, i3], axis=1)
    x = jnp.maximum(x, 0.0)
    r = lax.dot_general(
        x, w_ref[...], (((1,), (1,)), ((), ())),
        preferred_element_type=jnp.float32,
        precision=lax.Precision.HIGHEST) + b_ref[...]
    res_ref[...] = r
    p = jax.nn.sigmoid(r)
    v = jnp.dot(p, ddi_ref[...],
                preferred_element_type=jnp.float32,
                precision=lax.Precision.HIGHEST)
    bn_ref[...] = (0.0005 * jnp.sum(v * p))[None, None]


def _prep_idx(idx, fill=0):
    idx = idx.astype(jnp.int32).reshape(-1)
    return jnp.pad(idx, (0, L_PAD - L), constant_values=fill)


def kernel(diag_idx, proc_idx, med_idx, emb0, emb1, emb2, W, b, ddi_adj):
    i0 = _prep_idx(diag_idx).reshape(1, L_PAD)
    i1 = _prep_idx(proc_idx).reshape(1, L_PAD)
    med_col = _prep_idx(med_idx, fill=-1).reshape(L_PAD, 1)
    res, bn = pl.pallas_call(
        _body,
        in_specs=[
            pl.BlockSpec(memory_space=pltpu.MemorySpace.SMEM),
            pl.BlockSpec(memory_space=pltpu.MemorySpace.SMEM),
            pl.BlockSpec(memory_space=pltpu.MemorySpace.VMEM),
            pl.BlockSpec(memory_space=pltpu.MemorySpace.HBM),
            pl.BlockSpec(memory_space=pltpu.MemorySpace.HBM),
            pl.BlockSpec(memory_space=pltpu.MemorySpace.VMEM),
            pl.BlockSpec(memory_space=pltpu.MemorySpace.VMEM),
            pl.BlockSpec(memory_space=pltpu.MemorySpace.VMEM),
            pl.BlockSpec(memory_space=pltpu.MemorySpace.VMEM),
        ],
        out_shape=(jax.ShapeDtypeStruct((1, V2), jnp.float32),
                   jax.ShapeDtypeStruct((1, 1), jnp.float32)),
        scratch_shapes=[pltpu.VMEM((L_PAD, D), jnp.float32),
                        pltpu.VMEM((L_PAD, D), jnp.float32),
                        pltpu.SemaphoreType.DMA,
                        pltpu.SemaphoreType.DMA],
    )(i0, i1, med_col, emb0, emb1, emb2, W, b.reshape(1, V2), ddi_adj)
    return res, bn[0, 0]


# speedup vs baseline: 7.1248x; 4.0752x over previous
import jax
import jax.numpy as jnp
from jax import lax
from jax.experimental import pallas as pl
from jax.experimental.pallas import tpu as pltpu

L = 200
L_PAD = 256
D = 64
TW = 128
V2 = 1000


def _gather_sum(idx_smem, idx_vec_ref, et_hbm, tiles, sem):
    lane = lax.broadcasted_iota(jnp.int32, (L, 1, TW), 2)
    m = (lax.rem(idx_vec_ref[...], TW) == lane).astype(jnp.float32)
    sel = tiles[...] * m
    t = jnp.sum(sel, axis=2)
    return jnp.sum(t, axis=0, keepdims=True)


def _body(i0_ref, i1_ref, iv0_ref, iv1_ref, med_ref, e0t_hbm, e1t_hbm,
          e2t_ref, wt_ref, b_ref, ddi_ref, res_ref, bn_ref,
          tiles0, tiles1, sem0, sem1):
    def fire(i, _):
        s0 = pl.multiple_of((i0_ref[0, i] // TW) * TW, TW)
        pltpu.make_async_copy(e0t_hbm.at[:, pl.ds(s0, TW)],
                              tiles0.at[i], sem0).start()
        s1 = pl.multiple_of((i1_ref[0, i] // TW) * TW, TW)
        pltpu.make_async_copy(e1t_hbm.at[:, pl.ds(s1, TW)],
                              tiles1.at[i], sem1).start()
        return _
    lax.fori_loop(0, L, fire, None)

    iota = lax.broadcasted_iota(jnp.int32, (1, V2), 1)
    eq = (med_ref[...] == iota).astype(jnp.float32)
    counts = jnp.sum(eq, axis=0, keepdims=True)
    i3 = lax.dot_general(
        counts, e2t_ref[...], (((1,), (1,)), ((), ())),
        preferred_element_type=jnp.float32,
        precision=lax.Precision.HIGHEST)

    def drain(i, _):
        pltpu.make_async_copy(e0t_hbm.at[:, pl.ds(0, TW)],
                              tiles0.at[i], sem0).wait()
        pltpu.make_async_copy(e1t_hbm.at[:, pl.ds(0, TW)],
                              tiles1.at[i], sem1).wait()
        return _
    lax.fori_loop(0, L, drain, None)

    i1 = _gather_sum(i0_ref, iv0_ref, e0t_hbm, tiles0, sem0)
    i2 = _gather_sum(i1_ref, iv1_ref, e1t_hbm, tiles1, sem1)

    x = jnp.concatenate([i1, i2, i3], axis=1)
    x = jnp.maximum(x, 0.0)
    r = jnp.dot(x, wt_ref[...],
                preferred_element_type=jnp.float32,
                precision=lax.Precision.HIGHEST) + b_ref[...]
    res_ref[...] = r
    p = jax.nn.sigmoid(r)
    v = jnp.dot(p, ddi_ref[...],
                preferred_element_type=jnp.float32,
                precision=lax.Precision.HIGHEST)
    bn_ref[...] = (0.0005 * jnp.sum(v * p))[None, None]


def _prep_idx(idx, fill=0):
    idx = idx.astype(jnp.int32).reshape(-1)
    return jnp.pad(idx, (0, L_PAD - L), constant_values=fill)


def kernel(diag_idx, proc_idx, med_idx, emb0, emb1, emb2, W, b, ddi_adj):
    i0 = _prep_idx(diag_idx).reshape(1, L_PAD)
    i1 = _prep_idx(proc_idx).reshape(1, L_PAD)
    iv0 = diag_idx.astype(jnp.int32).reshape(L, 1, 1)
    iv1 = proc_idx.astype(jnp.int32).reshape(L, 1, 1)
    med_col = _prep_idx(med_idx, fill=-1).reshape(L_PAD, 1)
    res, bn = pl.pallas_call(
        _body,
        in_specs=[
            pl.BlockSpec(memory_space=pltpu.MemorySpace.SMEM),
            pl.BlockSpec(memory_space=pltpu.MemorySpace.SMEM),
            pl.BlockSpec(memory_space=pltpu.MemorySpace.VMEM),
            pl.BlockSpec(memory_space=pltpu.MemorySpace.VMEM),
            pl.BlockSpec(memory_space=pltpu.MemorySpace.VMEM),
            pl.BlockSpec(memory_space=pltpu.MemorySpace.HBM),
            pl.BlockSpec(memory_space=pltpu.MemorySpace.HBM),
            pl.BlockSpec(memory_space=pltpu.MemorySpace.VMEM),
            pl.BlockSpec(memory_space=pltpu.MemorySpace.VMEM),
            pl.BlockSpec(memory_space=pltpu.MemorySpace.VMEM),
            pl.BlockSpec(memory_space=pltpu.MemorySpace.VMEM),
        ],
        out_shape=(jax.ShapeDtypeStruct((1, V2), jnp.float32),
                   jax.ShapeDtypeStruct((1, 1), jnp.float32)),
        scratch_shapes=[pltpu.VMEM((L, D, TW), jnp.float32),
                        pltpu.VMEM((L, D, TW), jnp.float32),
                        pltpu.SemaphoreType.DMA,
                        pltpu.SemaphoreType.DMA],
    )(i0, i1, iv0, iv1, med_col, emb0.T, emb1.T, emb2.T, W.T,
      b.reshape(1, V2), ddi_adj)
    return res, bn[0, 0]


# speedup vs baseline: 9.3557x; 1.3131x over previous
import jax
import jax.numpy as jnp
from jax import lax
from jax.experimental import pallas as pl
from jax.experimental.pallas import tpu as pltpu

L = 200
D = 64
TW = 128
V2 = 1000


def _select_sum(idx_col, tiles):
    lane = lax.broadcasted_iota(jnp.int32, (L, 1, TW), 2)
    m = (idx_col[:, :, None] % TW == lane).astype(jnp.float32)
    sel = tiles[...] * m
    t = jnp.sum(sel, axis=2)
    return jnp.sum(t, axis=0, keepdims=True)


def _body(i0_ref, i1_ref, iv_ref, e0t_hbm, e1t_hbm,
          e2t_ref, wt_ref, b_ref, ddi_ref, res_ref, bn_ref,
          tiles0, tiles1, sem0, sem1):
    def fire(i, _):
        s0 = pl.multiple_of((i0_ref[0, i] // TW) * TW, TW)
        pltpu.make_async_copy(e0t_hbm.at[:, pl.ds(s0, TW)],
                              tiles0.at[i], sem0).start()
        s1 = pl.multiple_of((i1_ref[0, i] // TW) * TW, TW)
        pltpu.make_async_copy(e1t_hbm.at[:, pl.ds(s1, TW)],
                              tiles1.at[i], sem1).start()
        return _
    lax.fori_loop(0, L, fire, None)

    iv = iv_ref[...]
    iota = lax.broadcasted_iota(jnp.int32, (1, V2), 1)
    eq = (iv[:, 2:3] == iota).astype(jnp.float32)
    counts = jnp.sum(eq, axis=0, keepdims=True)
    i3 = lax.dot_general(
        counts, e2t_ref[...], (((1,), (1,)), ((), ())),
        preferred_element_type=jnp.float32,
        precision=lax.Precision.HIGHEST)

    def drain(i, _):
        pltpu.make_async_copy(e0t_hbm.at[:, pl.ds(0, TW)],
                              tiles0.at[i], sem0).wait()
        pltpu.make_async_copy(e1t_hbm.at[:, pl.ds(0, TW)],
                              tiles1.at[i], sem1).wait()
        return _
    lax.fori_loop(0, L, drain, None)

    i1 = _select_sum(iv[:, 0:1], tiles0)
    i2 = _select_sum(iv[:, 1:2], tiles1)

    x = jnp.concatenate([i1, i2, i3], axis=1)
    x = jnp.maximum(x, 0.0)
    r = jnp.dot(x, wt_ref[...],
                preferred_element_type=jnp.float32,
                precision=lax.Precision.HIGHEST) + b_ref[...]
    res_ref[...] = r
    p = jax.nn.sigmoid(r)
    v = jnp.dot(p, ddi_ref[...],
                preferred_element_type=jnp.float32,
                precision=lax.Precision.HIGHEST)
    bn_ref[...] = (0.0005 * jnp.sum(v * p))[None, None]


def kernel(diag_idx, proc_idx, med_idx, emb0, emb1, emb2, W, b, ddi_adj):
    i0 = diag_idx.astype(jnp.int32).reshape(1, L)
    i1 = proc_idx.astype(jnp.int32).reshape(1, L)
    iv = jnp.stack([diag_idx, proc_idx, med_idx], axis=1).astype(jnp.int32)
    res, bn = pl.pallas_call(
        _body,
        in_specs=[
            pl.BlockSpec(memory_space=pltpu.MemorySpace.SMEM),
            pl.BlockSpec(memory_space=pltpu.MemorySpace.SMEM),
            pl.BlockSpec(memory_space=pltpu.MemorySpace.VMEM),
            pl.BlockSpec(memory_space=pltpu.MemorySpace.HBM),
            pl.BlockSpec(memory_space=pltpu.MemorySpace.HBM),
            pl.BlockSpec(memory_space=pltpu.MemorySpace.VMEM),
            pl.BlockSpec(memory_space=pltpu.MemorySpace.VMEM),
            pl.BlockSpec(memory_space=pltpu.MemorySpace.VMEM),
            pl.BlockSpec(memory_space=pltpu.MemorySpace.VMEM),
        ],
        out_shape=(jax.ShapeDtypeStruct((1, V2), jnp.float32),
                   jax.ShapeDtypeStruct((1, 1), jnp.float32)),
        scratch_shapes=[pltpu.VMEM((L, D, TW), jnp.float32),
                        pltpu.VMEM((L, D, TW), jnp.float32),
                        pltpu.SemaphoreType.DMA,
                        pltpu.SemaphoreType.DMA],
    )(i0, i1, iv, emb0.T, emb1.T, emb2.T, W.T,
      b.reshape(1, V2), ddi_adj)
    return res, bn[0, 0]


# speedup vs baseline: 9.6404x; 1.0304x over previous
import jax
import jax.numpy as jnp
from jax import lax
from jax.experimental import pallas as pl
from jax.experimental.pallas import tpu as pltpu

L = 200
D = 64
TW = 128
V2 = 1000
NQ = 4
LQ = L // NQ


def _select_sum(idx_col, tiles):
    lane = lax.broadcasted_iota(jnp.int32, (L, 1, TW), 2)
    m = (idx_col[:, :, None] % TW == lane).astype(jnp.float32)
    sel = tiles[...] * m
    t = jnp.sum(sel, axis=2)
    return jnp.sum(t, axis=0, keepdims=True)


def _fire(idx_smem, et_hbm, tiles, sems):
    def go(i, _):
        for q in range(NQ):
            k = i * NQ + q
            s = pl.multiple_of((idx_smem[0, k] // TW) * TW, TW)
            pltpu.make_async_copy(et_hbm.at[:, pl.ds(s, TW)],
                                  tiles.at[k], sems[q]).start()
        return _
    lax.fori_loop(0, LQ, go, None)


def _drain(et_hbm, tiles, sems):
    def go(i, _):
        for q in range(NQ):
            k = i * NQ + q
            pltpu.make_async_copy(et_hbm.at[:, pl.ds(0, TW)],
                                  tiles.at[k], sems[q]).wait()
        return _
    lax.fori_loop(0, LQ, go, None)


def _body(i0_ref, i1_ref, iv_ref, e0t_hbm, e1t_hbm,
          e2t_ref, wt_ref, b_ref, ddi_ref, res_ref, bn_ref,
          tiles0, tiles1, *sems):
    _fire(i0_ref, e0t_hbm, tiles0, sems[:NQ])
    _fire(i1_ref, e1t_hbm, tiles1, sems[NQ:])

    iv = iv_ref[...]
    iota = lax.broadcasted_iota(jnp.int32, (1, V2), 1)
    eq = (iv[:, 2:3] == iota).astype(jnp.float32)
    counts = jnp.sum(eq, axis=0, keepdims=True)
    i3 = lax.dot_general(
        counts, e2t_ref[...], (((1,), (1,)), ((), ())),
        preferred_element_type=jnp.float32,
        precision=lax.Precision.HIGHEST)

    _drain(e0t_hbm, tiles0, sems[:NQ])
    i1 = _select_sum(iv[:, 0:1], tiles0)
    _drain(e1t_hbm, tiles1, sems[NQ:])
    i2 = _select_sum(iv[:, 1:2], tiles1)

    x = jnp.concatenate([i1, i2, i3], axis=1)
    x = jnp.maximum(x, 0.0)
    r = jnp.dot(x, wt_ref[...],
                preferred_element_type=jnp.float32,
                precision=lax.Precision.HIGHEST) + b_ref[...]
    res_ref[...] = r
    p = jax.nn.sigmoid(r)
    v = jnp.dot(p, ddi_ref[...],
                preferred_element_type=jnp.float32,
                precision=lax.Precision.HIGHEST)
    bn_ref[...] = (0.0005 * jnp.sum(v * p))[None, None]


def kernel(diag_idx, proc_idx, med_idx, emb0, emb1, emb2, W, b, ddi_adj):
    i0 = diag_idx.astype(jnp.int32).reshape(1, L)
    i1 = proc_idx.astype(jnp.int32).reshape(1, L)
    iv = jnp.stack([diag_idx, proc_idx, med_idx], axis=1).astype(jnp.int32)
    res, bn = pl.pallas_call(
        _body,
        in_specs=[
            pl.BlockSpec(memory_space=pltpu.MemorySpace.SMEM),
            pl.BlockSpec(memory_space=pltpu.MemorySpace.SMEM),
            pl.BlockSpec(memory_space=pltpu.MemorySpace.VMEM),
            pl.BlockSpec(memory_space=pltpu.MemorySpace.HBM),
            pl.BlockSpec(memory_space=pltpu.MemorySpace.HBM),
            pl.BlockSpec(memory_space=pltpu.MemorySpace.VMEM),
            pl.BlockSpec(memory_space=pltpu.MemorySpace.VMEM),
            pl.BlockSpec(memory_space=pltpu.MemorySpace.VMEM),
            pl.BlockSpec(memory_space=pltpu.MemorySpace.VMEM),
        ],
        out_shape=(jax.ShapeDtypeStruct((1, V2), jnp.float32),
                   jax.ShapeDtypeStruct((1, 1), jnp.float32)),
        scratch_shapes=([pltpu.VMEM((L, D, TW), jnp.float32),
                         pltpu.VMEM((L, D, TW), jnp.float32)]
                        + [pltpu.SemaphoreType.DMA] * (2 * NQ)),
    )(i0, i1, iv, emb0.T, emb1.T, emb2.T, W.T,
      b.reshape(1, V2), ddi_adj)
    return res, bn[0, 0]
